# baseline (speedup 1.0000x reference)
import functools
import jax
import jax.numpy as jnp
from jax import lax
from jax.experimental import pallas as pl
from jax.experimental.pallas import tpu as pltpu
from jax.experimental.pallas import tpu_sc as plsc

DROP_P = 0.1
KEEP = 1.0 - DROP_P
SCALE = 1.0 / KEEP

NC = 2
NS = 16
NW = NC * NS
L = 16

B = 4096 * 50
D = 64
CH = 128
BPW = B // NW
NCHUNK = BPW // CH


def _sc_body(w_hbm, u_hbm, x_hbm, out_hbm,
             idx_v, idxh_v, off_v, u_v, rows_v, out_v, sem_u, sem_r):
    cid = lax.axis_index("c")
    sid = lax.axis_index("s")
    wid = sid * NC + cid
    pltpu.sync_copy(x_hbm.at[pl.ds(wid * BPW, BPW)], idx_v)
    out_base = wid * BPW

    def prep(i, carry):
        for t in range(8):
            sl = pl.ds((i * 8 + t) * L, L)
            iv = idx_v[sl]
            idxh_v[sl] = lax.shift_right_logical(iv, 1)
            off_v[sl] = lax.shift_left(jnp.bitwise_and(iv, 1), 6)
        return carry

    lax.fori_loop(0, BPW // (8 * L), prep, 0)

    def chunk(j, carry):
        cp_u = pltpu.async_copy(u_hbm.at[idx_v.at[pl.ds(j * CH, CH)]], u_v, sem_u)
        cp_r = pltpu.async_copy(w_hbm.at[idxh_v.at[pl.ds(j * CH, CH)]], rows_v, sem_r)
        cp_u.wait()
        cp_r.wait()
        for g in range(CH // L):
            u16 = u_v[pl.ds(g * L, L)]
            s16 = jnp.where(u16 < KEEP, jnp.float32(SCALE), jnp.float32(0.0))
            o16 = off_v[pl.ds(j * CH + g * L, L)]
            for r in range(L):
                row = g * L + r
                sv = jnp.full((L,), s16[r], jnp.float32)
                off = o16[r]
                for cg in range(D // L):
                    out_v[row, pl.ds(cg * L, L)] = (
                        rows_v[row, pl.ds(off + cg * L, L)] * sv
                    )
        pltpu.sync_copy(out_v, out_hbm.at[pl.ds(out_base + j * CH, CH)])
        return carry

    lax.fori_loop(0, NCHUNK, chunk, 0)


@jax.jit
def _embedding_dropout(x_flat, weight2, u_flat):
    mesh = plsc.VectorSubcoreMesh(
        core_axis_name="c", subcore_axis_name="s", num_cores=NC, num_subcores=NS
    )
    fn = pl.kernel(
        _sc_body,
        out_type=jax.ShapeDtypeStruct((B, D), jnp.float32),
        mesh=mesh,
        scratch_types=[
            pltpu.VMEM((BPW,), jnp.int32),
            pltpu.VMEM((BPW,), jnp.int32),
            pltpu.VMEM((BPW,), jnp.int32),
            pltpu.VMEM((CH,), jnp.float32),
            pltpu.VMEM((CH, 2 * D), jnp.float32),
            pltpu.VMEM((CH, D), jnp.float32),
            pltpu.SemaphoreType.DMA,
            pltpu.SemaphoreType.DMA,
        ],
        compiler_params=pltpu.CompilerParams(use_tc_tiling_on_sc=True),
    )
    return fn(weight2, u_flat, x_flat)


def kernel(x, weight, row_mask_u):
    x_flat = x.reshape(-1).astype(jnp.int32)
    u_flat = row_mask_u.reshape(-1)
    weight2 = weight.reshape(weight.shape[0] // 2, 2 * D)
    out = _embedding_dropout(x_flat, weight2, u_flat)
    return out.reshape(x.shape[0], x.shape[1], D)
